# baseline (speedup 1.0000x reference)
import jax
import jax.numpy as jnp
from jax.experimental import pallas as pl
from jax.experimental.pallas import tpu as pltpu

_DIM = 384
_DF = 24
_WS = 7
_K = 9
_PATCH = 32
_HALF = 16
_N = _PATCH * _PATCH
_NH = _PATCH * _HALF


def _fused_kernel(
    x_ref, fw_ref, fb_ref, pw_ref, pb_ref, ab_ref, o_hbm, fnt_all, pixbuf, osem
):
    i = pl.program_id(0)
    j = pl.program_id(1)
    fw = fw_ref[...]
    fb = fb_ref[...]

    for jj in range(2):

        @pl.when(j == jj)
        def _stage():
            for h in range(_WS):
                xblk = x_ref[:, :, h * _PATCH : (h + 1) * _PATCH].reshape(
                    _DIM, _NH
                )
                fnt_all[h, :, jj * _NH : (jj + 1) * _NH] = (
                    jnp.dot(fw, xblk, preferred_element_type=jnp.float32) + fb
                )

    @pl.when(j == 1)
    def _graph():
        pw = pw_ref[...]
        pb = pb_ref[...]
        ab = ab_ref[...]
        alpha = ab[:, 0:1]
        beta = ab[:, 1:2]

        def body(h, carry):
            fnT = fnt_all[h]
            n2 = jnp.sum(fnT * fnT, axis=0, keepdims=True)
            inv = 1.0 / jnp.maximum(jnp.sqrt(n2), 1e-8)
            xnT = fnT * inv
            sim = jax.lax.dot_general(
                xnT, xnT, (((0,), (0,)), ((), ())),
                preferred_element_type=jnp.float32,
            )
            neg = jnp.float32(-jnp.inf)
            m = jnp.max(sim, axis=0, keepdims=True)
            for _ in range(_K - 1):
                m = jnp.max(
                    jnp.where(sim < m, sim, neg), axis=0, keepdims=True
                )
            w = jnp.where(sim >= m, jax.nn.sigmoid(beta + alpha * sim), 0.0)
            xe = jnp.concatenate(
                [fnT, jnp.ones((1, _N), jnp.float32)], axis=0
            )
            agg = jnp.dot(xe, w, preferred_element_type=jnp.float32)
            outT = agg[:_DF] / (agg[_DF : _DF + 1] + 1e-12)
            pix = jnp.dot(pw, outT, preferred_element_type=jnp.float32) + pb

            b = jax.lax.rem(h, 2)

            @pl.when((h >= 2) | (i >= 1))
            def _wait_prev():
                pltpu.make_async_copy(
                    pixbuf.at[b], o_hbm.at[:, 0, :, 0, :], osem.at[b]
                ).wait()

            pixbuf[b] = pix.reshape(_DIM, _PATCH, _PATCH)
            pltpu.make_async_copy(
                pixbuf.at[b], o_hbm.at[:, i, :, h, :], osem.at[b]
            ).start()
            return carry

        jax.lax.fori_loop(0, _WS, body, 0)

        @pl.when(i == _WS - 1)
        def _drain():
            pltpu.make_async_copy(
                pixbuf.at[1], o_hbm.at[:, 0, :, 0, :], osem.at[1]
            ).wait()
            pltpu.make_async_copy(
                pixbuf.at[0], o_hbm.at[:, 0, :, 0, :], osem.at[0]
            ).wait()


def kernel(x_in, f_w, f_b, p_w, p_b, edge_alpha, edge_beta):
    B, C, H, W = x_in.shape
    x3 = x_in.reshape(C, H, W)
    ab = jnp.concatenate([edge_alpha, edge_beta]).reshape(1, 2)

    out = pl.pallas_call(
        _fused_kernel,
        grid=(_WS, 2),
        in_specs=[
            pl.BlockSpec((C, _HALF, W), lambda i, j: (0, 2 * i + j, 0)),
            pl.BlockSpec((_DF, C), lambda i, j: (0, 0)),
            pl.BlockSpec((_DF, 1), lambda i, j: (0, 0)),
            pl.BlockSpec((C, _DF), lambda i, j: (0, 0)),
            pl.BlockSpec((C, 1), lambda i, j: (0, 0)),
            pl.BlockSpec((1, 2), lambda i, j: (0, 0)),
        ],
        out_specs=pl.BlockSpec(memory_space=pl.ANY),
        out_shape=jax.ShapeDtypeStruct(
            (C, _WS, _PATCH, _WS, _PATCH), jnp.float32
        ),
        scratch_shapes=[
            pltpu.VMEM((_WS, _DF, _N), jnp.float32),
            pltpu.VMEM((2, _DIM, _PATCH, _PATCH), jnp.float32),
            pltpu.SemaphoreType.DMA((2,)),
        ],
    )(x3, f_w, f_b.reshape(_DF, 1), p_w, p_b.reshape(C, 1), ab)

    return out.reshape(B, C, H * W)


# speedup vs baseline: 1.5369x; 1.5369x over previous
import jax
import jax.numpy as jnp
from jax.experimental import pallas as pl
from jax.experimental.pallas import tpu as pltpu

_DIM = 384
_DF = 24
_WS = 7
_K = 9
_PATCH = 32
_HALF = 16
_N = _PATCH * _PATCH
_NH = _PATCH * _HALF
_HW = 224 * 224
_CHUNK = 3584
_NCHUNK = _HW // _CHUNK


def _graph_kernel(x_ref, fw_ref, fb_ref, ab_ref, o_ref, fnt_all):
    j = pl.program_id(1)
    fw = fw_ref[...]
    fb = fb_ref[...]

    for jj in range(2):

        @pl.when(j == jj)
        def _stage():
            for h in range(_WS):
                xblk = x_ref[:, :, h * _PATCH : (h + 1) * _PATCH].reshape(
                    _DIM, _NH
                )
                fnt_all[h, :, jj * _NH : (jj + 1) * _NH] = (
                    jnp.dot(fw, xblk, preferred_element_type=jnp.float32) + fb
                )

    @pl.when(j == 1)
    def _graph():
        ab = ab_ref[...]
        alpha = ab[:, 0:1]
        beta = ab[:, 1:2]

        def body(h, carry):
            fnT = fnt_all[h]
            n2 = jnp.sum(fnT * fnT, axis=0, keepdims=True)
            inv = 1.0 / jnp.maximum(jnp.sqrt(n2), 1e-8)
            xnT = fnT * inv
            sim = jax.lax.dot_general(
                xnT, xnT, (((0,), (0,)), ((), ())),
                preferred_element_type=jnp.float32,
            )
            neg = jnp.float32(-jnp.inf)
            m = jnp.max(sim, axis=0, keepdims=True)
            for _ in range(_K - 1):
                m = jnp.max(
                    jnp.where(sim < m, sim, neg), axis=0, keepdims=True
                )
            w = jnp.where(sim >= m, jax.nn.sigmoid(beta + alpha * sim), 0.0)
            xe = jnp.concatenate(
                [fnT, jnp.ones((1, _N), jnp.float32)], axis=0
            )
            agg = jnp.dot(xe, w, preferred_element_type=jnp.float32)
            o_ref[h] = agg[:_DF] / (agg[_DF : _DF + 1] + 1e-12)
            return carry

        jax.lax.fori_loop(0, _WS, body, 0)


def _proj_kernel(x_ref, w_ref, b_ref, o_ref):
    o_ref[...] = (
        jnp.dot(w_ref[...], x_ref[...], preferred_element_type=jnp.float32)
        + b_ref[...]
    )


def kernel(x_in, f_w, f_b, p_w, p_b, edge_alpha, edge_beta):
    B, C, H, W = x_in.shape
    x3 = x_in.reshape(C, H, W)
    ab = jnp.concatenate([edge_alpha, edge_beta]).reshape(1, 2)

    out_nodes = pl.pallas_call(
        _graph_kernel,
        grid=(_WS, 2),
        in_specs=[
            pl.BlockSpec((C, _HALF, W), lambda i, j: (0, 2 * i + j, 0)),
            pl.BlockSpec((_DF, C), lambda i, j: (0, 0)),
            pl.BlockSpec((_DF, 1), lambda i, j: (0, 0)),
            pl.BlockSpec((1, 2), lambda i, j: (0, 0)),
        ],
        out_specs=pl.BlockSpec((_WS, _DF, _N), lambda i, j: (i, 0, 0)),
        out_shape=jax.ShapeDtypeStruct((_WS * _WS, _DF, _N), jnp.float32),
        scratch_shapes=[pltpu.VMEM((_WS, _DF, _N), jnp.float32)],
    )(x3, f_w, f_b.reshape(_DF, 1), ab)

    out_pix = (
        out_nodes.reshape(_WS, _WS, _DF, _PATCH, _PATCH)
        .transpose(2, 0, 3, 1, 4)
        .reshape(_DF, _HW)
    )

    out = pl.pallas_call(
        _proj_kernel,
        grid=(_NCHUNK,),
        in_specs=[
            pl.BlockSpec((_DF, _CHUNK), lambda i: (0, i)),
            pl.BlockSpec((C, _DF), lambda i: (0, 0)),
            pl.BlockSpec((C, 1), lambda i: (0, 0)),
        ],
        out_specs=pl.BlockSpec((C, _CHUNK), lambda i: (0, i)),
        out_shape=jax.ShapeDtypeStruct((C, _HW), jnp.float32),
    )(out_pix, p_w, p_b.reshape(C, 1))

    return out.reshape(B, C, _HW)


# speedup vs baseline: 1.9386x; 1.2614x over previous
import jax
import jax.numpy as jnp
from jax.experimental import pallas as pl

_DIM = 384
_DF = 24
_WS = 7
_K = 9
_PATCH = 32
_N = _PATCH * _PATCH
_NP = _WS * _WS
_HW = 224 * 224
_CHUNK = 3584
_NCHUNK = _HW // _CHUNK


def _proj_kernel(x_ref, w_ref, b_ref, o_ref):
    o_ref[...] = (
        jnp.dot(w_ref[...], x_ref[...], preferred_element_type=jnp.float32)
        + b_ref[...]
    )


def _graph_kernel(fn_ref, ab_ref, o_ref):
    x = fn_ref[0]
    ab = ab_ref[...]
    alpha = ab[:, 0:1]
    beta = ab[:, 1:2]
    n2 = jnp.sum(x * x, axis=1, keepdims=True)
    inv = 1.0 / jnp.maximum(jnp.sqrt(n2), 1e-8)
    xn = x * inv
    sim = jax.lax.dot_general(
        xn, xn, (((1,), (1,)), ((), ())), preferred_element_type=jnp.float32
    )
    neg = jnp.float32(-jnp.inf)
    m = jnp.max(sim, axis=1, keepdims=True)
    for _ in range(_K - 1):
        m = jnp.max(jnp.where(sim < m, sim, neg), axis=1, keepdims=True)
    w = jnp.where(sim >= m, jax.nn.sigmoid(beta + alpha * sim), 0.0)
    xe = jnp.concatenate([x, jnp.ones((_N, 1), jnp.float32)], axis=1)
    agg = jnp.dot(w, xe, preferred_element_type=jnp.float32)
    o_ref[0] = agg[:, :_DF] / (agg[:, _DF:_DF + 1] + 1e-12)


def kernel(x_in, f_w, f_b, p_w, p_b, edge_alpha, edge_beta):
    B, C, H, W = x_in.shape
    x2 = x_in.reshape(C, _HW)
    ab = jnp.concatenate([edge_alpha, edge_beta]).reshape(1, 2)

    f_pix = pl.pallas_call(
        _proj_kernel,
        grid=(_NCHUNK,),
        in_specs=[
            pl.BlockSpec((C, _CHUNK), lambda i: (0, i)),
            pl.BlockSpec((_DF, C), lambda i: (0, 0)),
            pl.BlockSpec((_DF, 1), lambda i: (0, 0)),
        ],
        out_specs=pl.BlockSpec((_DF, _CHUNK), lambda i: (0, i)),
        out_shape=jax.ShapeDtypeStruct((_DF, _HW), jnp.float32),
    )(x2, f_w, f_b.reshape(_DF, 1))

    f_nodes = (
        f_pix.reshape(_DF, _WS, _PATCH, _WS, _PATCH)
        .transpose(1, 3, 2, 4, 0)
        .reshape(_NP, _N, _DF)
    )

    out_nodes = pl.pallas_call(
        _graph_kernel,
        grid=(_NP,),
        in_specs=[
            pl.BlockSpec((1, _N, _DF), lambda p: (p, 0, 0)),
            pl.BlockSpec((1, 2), lambda p: (0, 0)),
        ],
        out_specs=pl.BlockSpec((1, _N, _DF), lambda p: (p, 0, 0)),
        out_shape=jax.ShapeDtypeStruct((_NP, _N, _DF), jnp.float32),
    )(f_nodes, ab)

    out_pix = (
        out_nodes.reshape(_WS, _WS, _PATCH, _PATCH, _DF)
        .transpose(4, 0, 2, 1, 3)
        .reshape(_DF, _HW)
    )

    out = pl.pallas_call(
        _proj_kernel,
        grid=(_NCHUNK,),
        in_specs=[
            pl.BlockSpec((_DF, _CHUNK), lambda i: (0, i)),
            pl.BlockSpec((C, _DF), lambda i: (0, 0)),
            pl.BlockSpec((C, 1), lambda i: (0, 0)),
        ],
        out_specs=pl.BlockSpec((C, _CHUNK), lambda i: (0, i)),
        out_shape=jax.ShapeDtypeStruct((C, _HW), jnp.float32),
    )(out_pix, p_w, p_b.reshape(C, 1))

    return out.reshape(B, C, _HW)
